# baseline (speedup 1.0000x reference)
import functools

import jax
import jax.numpy as jnp
from jax import lax
from jax.experimental import pallas as pl
from jax.experimental.pallas import tpu as pltpu
from jax.experimental.pallas import tpu_sc as plsc

NC = 2
NS = 16
LANES = 16
CHUNK = 48
ZROWS = 40


def _z_body(x_ref, wm_ref, wl_ref, nt_ref, z_ref):
    xb = x_ref[...]
    dn = (((1,), (1,)), ((), ()))
    zm = lax.dot_general(xb, wm_ref[...], dn,
                         preferred_element_type=jnp.float32,
                         precision=lax.Precision.HIGHEST)
    zl = lax.dot_general(xb, wl_ref[...], dn,
                         preferred_element_type=jnp.float32,
                         precision=lax.Precision.HIGHEST)
    z_ref[...] = jnp.where(nt_ref[...] == 0, zm, zl)


def _compute_z(x, wm, wl, nt2d):
    n, d = x.shape
    a = wm.shape[0]
    blk = 2000
    return pl.pallas_call(
        _z_body,
        grid=(n // blk,),
        in_specs=[
            pl.BlockSpec((blk, d), lambda i: (i, 0)),
            pl.BlockSpec((a, d), lambda i: (0, 0)),
            pl.BlockSpec((a, d), lambda i: (0, 0)),
            pl.BlockSpec((blk, 1), lambda i: (i, 0)),
        ],
        out_specs=pl.BlockSpec((blk, a), lambda i: (i, 0)),
        out_shape=jax.ShapeDtypeStruct((n, a), jnp.float32),
    )(x, wm, wl, nt2d)


def _edge_pass(src3d, dst3d, z):
    n, a = z.shape
    ntiles, nch, _ = src3d.shape
    npad = ((n + NS * 128 - 1) // (NS * 128)) * (NS * 128)
    rows_per_tile = npad // NS
    dr = npad // 128

    mesh = plsc.VectorSubcoreMesh(core_axis_name="c", subcore_axis_name="s")

    @functools.partial(
        pl.kernel,
        out_type=(jax.ShapeDtypeStruct((NC, npad, a), jnp.float32),
                  jax.ShapeDtypeStruct((NC, dr, 128), jnp.float32)),
        mesh=mesh,
        compiler_params=pltpu.CompilerParams(needs_layout_passes=False),
        scratch_types=[
            pltpu.VMEM((CHUNK,), jnp.int32),
            pltpu.VMEM((CHUNK,), jnp.int32),
            pltpu.VMEM((CHUNK,), jnp.int32),
            pltpu.VMEM((CHUNK,), jnp.int32),
            pltpu.VMEM((CHUNK,), jnp.int32),
            pltpu.VMEM((CHUNK,), jnp.int32),
            pltpu.VMEM((CHUNK, a), jnp.float32),
            pltpu.VMEM((CHUNK, a), jnp.float32),
            pltpu.VMEM((CHUNK, a), jnp.float32),
            pltpu.VMEM((CHUNK, a), jnp.float32),
            pltpu.VMEM((CHUNK, a), jnp.float32),
            pltpu.VMEM((CHUNK, a), jnp.float32),
            pltpu.VMEM((dr, 128), jnp.float32),
            pltpu.VMEM((dr,), jnp.int32),
            pltpu.VMEM_SHARED((npad, a), jnp.float32),
            pltpu.VMEM_SHARED((dr, 128), jnp.float32),
            pltpu.SemaphoreType.DMA,
            pltpu.SemaphoreType.DMA,
            pltpu.SemaphoreType.DMA,
            pltpu.SemaphoreType.DMA,
            pltpu.SemaphoreType.DMA,
            pltpu.SemaphoreType.DMA,
            pltpu.SemaphoreType.DMA,
        ],
    )
    def edge_kernel(src_hbm, dst_hbm, z_hbm, outh_hbm, outd_hbm,
                    ixs0, ixs1, ixd0, ixd1, ixc0, ixc1,
                    zs0, zs1, zd0, zd1, st0, st1,
                    den, idr, acc, accd,
                    gsem0, gsem1, ssem0, ssem1, isem0, isem1, zsem):
        c = lax.axis_index("c")
        s = lax.axis_index("s")
        wid = c * NS + s
        iota = lax.iota(jnp.int32, LANES)
        zeros = jnp.zeros((LANES,), jnp.float32)
        ixs, ixd, ixc = (ixs0, ixs1), (ixd0, ixd1), (ixc0, ixc1)
        zsb, zdb, stb = (zs0, zs1), (zd0, zd1), (st0, st1)
        gsem, ssem, isem = (gsem0, gsem1), (ssem0, ssem1), (isem0, isem1)

        def issue_idx(cid, b):
            pltpu.async_copy(src_hbm.at[wid].at[cid], ixs[b], isem[b])
            pltpu.async_copy(dst_hbm.at[wid].at[cid], ixd[b], isem[b])

        def wait_idx(cid, b):
            pltpu.make_async_copy(
                src_hbm.at[wid].at[cid], ixs[b], isem[b]).wait()
            pltpu.make_async_copy(
                dst_hbm.at[wid].at[cid], ixd[b], isem[b]).wait()

        def issue_gather(cid, b):
            pltpu.async_copy(z_hbm.at[ixs[b]], zsb[b], gsem[b])
            pltpu.async_copy(z_hbm.at[ixd[b]], zdb[b], gsem[b])

        def wait_gather(cid, b):
            pltpu.make_async_copy(z_hbm.at[ixs[b]], zsb[b], gsem[b]).wait()
            pltpu.make_async_copy(z_hbm.at[ixd[b]], zdb[b], gsem[b]).wait()

        def issue_scatter(cid, b):
            pltpu.async_copy(stb[b], acc.at[ixc[b]], ssem[b], add=True)

        def wait_scatter(cid, b):
            pltpu.make_async_copy(stb[b], acc.at[ixc[b]], ssem[b]).wait()

        issue_idx(0, 0)
        issue_idx(1, 1)

        @pl.loop(0, dr)
        def _(r):
            @pl.loop(0, 128, step=LANES)
            def _(j):
                den[r, pl.ds(j, LANES)] = zeros

        @pl.loop(0, dr, step=LANES)
        def _(r):
            idr[pl.ds(r, LANES)] = iota + r


        @pl.loop(0, ZROWS)
        def _(r):
            @pl.loop(0, 128, step=LANES)
            def _(j):
                st0[r, pl.ds(j, LANES)] = zeros

        @pl.loop(0, rows_per_tile, step=ZROWS)
        def _(r):
            pltpu.async_copy(st0.at[pl.ds(0, ZROWS)],
                             acc.at[pl.ds(s * rows_per_tile + r, ZROWS)],
                             zsem)

        @pl.when(s * 8 < dr)
        def _():
            pltpu.sync_copy(st0.at[pl.ds(0, 8)], accd.at[pl.ds(s * 8, 8)])

        @pl.loop(0, rows_per_tile, step=ZROWS)
        def _(r):
            pltpu.make_async_copy(
                st0.at[pl.ds(0, ZROWS)],
                acc.at[pl.ds(s * rows_per_tile + r, ZROWS)], zsem).wait()

        plsc.subcore_barrier()

        wait_idx(0, 0)
        issue_gather(0, 0)

        def compute(cid, b):
            zs, zd, stage, ixcb = zsb[b], zdb[b], stb[b], ixc[b]

            for g in range(0, CHUNK, LANES):
                idxv = ixcb[pl.ds(g, LANES)]

                def tree(vals):
                    while len(vals) > 1:
                        vals = [vals[k] + vals[k + 1]
                                for k in range(0, len(vals) - 1, 2)] + (
                            [vals[-1]] if len(vals) % 2 else [])
                    return vals[0]

                NI = 4
                for l in range(0, LANES, NI):
                    ii = [g + l + t for t in range(NI)]
                    zsv, tot = [], []
                    for t in range(NI):
                        zrow = [zs[ii[t], pl.ds(16 * j, 16)]
                                for j in range(a // 16)]
                        drow = [zd[ii[t], pl.ds(16 * j, 16)]
                                for j in range(a // 16)]
                        zsv.append(zrow)
                        tot.append(tree([zrow[j] * drow[j]
                                         for j in range(a // 16)]))
                    tot = [(plsc.cumsum(v)
                            + jnp.flip(plsc.cumsum(jnp.flip(v)))) - v
                           for v in tot]
                    tot = [jnp.where(v > 0.0, v, 0.2 * v) for v in tot]
                    exv = [jnp.exp(jnp.minimum(v, 80.0)) for v in tot]
                    for t in range(NI):
                        for j in range(a // 16):
                            stage[ii[t], pl.ds(16 * j, 16)] = (
                                exv[t] * zsv[t][j])
                    for t in range(NI):
                        di = idxv[l + t]
                        r0 = lax.shift_right_logical(di, 7)
                        c0 = jnp.bitwise_and(di, 0x70)
                        lane = jnp.bitwise_and(di, 0xF)
                        upd = jnp.where(iota == lane, exv[t], 0.0)
                        den[r0, pl.ds(c0, LANES)] = (
                            den[r0, pl.ds(c0, LANES)] + upd)

        @pl.loop(0, ((nch + 1) // 2) * 2, step=2)
        def _(base_cid):
            for b in range(2):
                cid = base_cid + b

                def chunk_body():
                    wait_gather(cid, b)

                    @pl.when(cid >= 2)
                    def _():
                        wait_scatter(cid - 2, b)

                    @pl.loop(0, CHUNK, step=LANES)
                    def _(g):
                        ixc[b][pl.ds(g, LANES)] = ixd[b][pl.ds(g, LANES)]

                    @pl.when(cid + 2 < nch)
                    def _():
                        issue_idx(cid + 2, b)

                    @pl.when(cid + 1 < nch)
                    def _():
                        wait_idx(cid + 1, 1 - b)
                        issue_gather(cid + 1, 1 - b)

                    compute(cid, b)
                    issue_scatter(cid, b)

                if b == 0:
                    chunk_body()
                else:
                    pl.when(cid < nch)(chunk_body)

        wait_scatter(nch - 1, (nch - 1) % 2)
        wait_scatter(nch - 2, (nch - 2) % 2)

        pltpu.sync_copy(den, accd.at[idr], add=True)
        plsc.subcore_barrier()

        r0 = s * rows_per_tile
        pltpu.sync_copy(acc.at[pl.ds(r0, rows_per_tile)],
                        outh_hbm.at[c].at[pl.ds(r0, rows_per_tile)])

        @pl.when(s * 8 < dr)
        def _():
            pltpu.sync_copy(accd.at[pl.ds(s * 8, 8)],
                            outd_hbm.at[c].at[pl.ds(s * 8, 8)])

    return edge_kernel(src3d, dst3d, z)


def _finish_body(hs_ref, den_ref, out_ref):
    h = hs_ref[0] + hs_ref[1]
    den = den_ref[0] + den_ref[1]
    v = h / (den + 1e-16)
    out_ref[...] = jnp.where(v > 0, v, jnp.exp(jnp.minimum(v, 0.0)) - 1.0)


def _finish(hs, den, npad, a):
    blk = 2048
    return pl.pallas_call(
        _finish_body,
        grid=(npad // blk,),
        in_specs=[
            pl.BlockSpec((NC, blk, a), lambda i: (0, i, 0)),
            pl.BlockSpec((NC, blk, 1), lambda i: (0, i, 0)),
        ],
        out_specs=pl.BlockSpec((blk, a), lambda i: (i, 0)),
        out_shape=jax.ShapeDtypeStruct((npad, a), jnp.float32),
    )(hs, den)


def kernel(x, Wm, Wl, edge_index, node_type):
    n, d = x.shape
    a = Wm.shape[0]
    e = edge_index.shape[1]
    ntiles = NC * NS
    nch = -(-e // (ntiles * CHUNK))
    epad = ntiles * nch * CHUNK - e
    src = edge_index[0]
    dst = edge_index[1]
    if epad:
        fill = jnp.arange(epad, dtype=jnp.int32)
        src = jnp.concatenate([src, fill % n])
        dst = jnp.concatenate([dst, n + (fill % 128)])
    src3d = src.reshape(ntiles, nch, CHUNK)
    dst3d = dst.reshape(ntiles, nch, CHUNK)
    z = _compute_z(x, Wm, Wl, node_type.reshape(n, 1))
    hs, dend = _edge_pass(src3d, dst3d, z)
    npad = hs.shape[1]
    den = dend.reshape(NC, npad, 1)
    out = _finish(hs, den, npad, a)
    return out[:n]


# speedup vs baseline: 2.0813x; 2.0813x over previous
import functools

import jax
import jax.numpy as jnp
from jax import lax
from jax.experimental import pallas as pl
from jax.experimental.pallas import tpu as pltpu
from jax.experimental.pallas import tpu_sc as plsc

NC = 2
NS = 16
LANES = 16
CHUNK = 48
ZROWS = 40


def _z_body(x_ref, wm_ref, wl_ref, nt_ref, z_ref):
    xb = x_ref[...]
    dn = (((1,), (1,)), ((), ()))
    zm = lax.dot_general(xb, wm_ref[...], dn,
                         preferred_element_type=jnp.float32,
                         precision=lax.Precision.DEFAULT)
    zl = lax.dot_general(xb, wl_ref[...], dn,
                         preferred_element_type=jnp.float32,
                         precision=lax.Precision.DEFAULT)
    z_ref[...] = jnp.where(nt_ref[...] == 0, zm, zl)


def _compute_z(x, wm, wl, nt2d):
    n, d = x.shape
    a = wm.shape[0]
    blk = 2000
    return pl.pallas_call(
        _z_body,
        grid=(n // blk,),
        in_specs=[
            pl.BlockSpec((blk, d), lambda i: (i, 0)),
            pl.BlockSpec((a, d), lambda i: (0, 0)),
            pl.BlockSpec((a, d), lambda i: (0, 0)),
            pl.BlockSpec((blk, 1), lambda i: (i, 0)),
        ],
        out_specs=pl.BlockSpec((blk, a), lambda i: (i, 0)),
        out_shape=jax.ShapeDtypeStruct((n, a), jnp.float32),
    )(x, wm, wl, nt2d)


def _edge_pass(src3d, dst3d, z):
    n, a = z.shape
    ntiles, nch, _ = src3d.shape
    npad = ((n + NS * 128 - 1) // (NS * 128)) * (NS * 128)
    rows_per_tile = npad // NS
    dr = npad // 128

    mesh = plsc.VectorSubcoreMesh(core_axis_name="c", subcore_axis_name="s")

    @functools.partial(
        pl.kernel,
        out_type=(jax.ShapeDtypeStruct((NC, npad, a), jnp.float32),
                  jax.ShapeDtypeStruct((NC, dr, 128), jnp.float32)),
        mesh=mesh,
        compiler_params=pltpu.CompilerParams(needs_layout_passes=False),
        scratch_types=[
            pltpu.VMEM((CHUNK,), jnp.int32),
            pltpu.VMEM((CHUNK,), jnp.int32),
            pltpu.VMEM((CHUNK,), jnp.int32),
            pltpu.VMEM((CHUNK,), jnp.int32),
            pltpu.VMEM((CHUNK,), jnp.int32),
            pltpu.VMEM((CHUNK,), jnp.int32),
            pltpu.VMEM((CHUNK, a), jnp.float32),
            pltpu.VMEM((CHUNK, a), jnp.float32),
            pltpu.VMEM((CHUNK, a), jnp.float32),
            pltpu.VMEM((CHUNK, a), jnp.float32),
            pltpu.VMEM((CHUNK, a), jnp.float32),
            pltpu.VMEM((CHUNK, a), jnp.float32),
            pltpu.VMEM((dr, 128), jnp.float32),
            pltpu.VMEM((dr,), jnp.int32),
            pltpu.VMEM_SHARED((npad, a), jnp.float32),
            pltpu.VMEM_SHARED((dr, 128), jnp.float32),
            pltpu.SemaphoreType.DMA,
            pltpu.SemaphoreType.DMA,
            pltpu.SemaphoreType.DMA,
            pltpu.SemaphoreType.DMA,
            pltpu.SemaphoreType.DMA,
            pltpu.SemaphoreType.DMA,
            pltpu.SemaphoreType.DMA,
        ],
    )
    def edge_kernel(src_hbm, dst_hbm, z_hbm, outh_hbm, outd_hbm,
                    ixs0, ixs1, ixd0, ixd1, ixc0, ixc1,
                    zs0, zs1, zd0, zd1, st0, st1,
                    den, idr, acc, accd,
                    gsem0, gsem1, ssem0, ssem1, isem0, isem1, zsem):
        c = lax.axis_index("c")
        s = lax.axis_index("s")
        wid = c * NS + s
        iota = lax.iota(jnp.int32, LANES)
        zeros = jnp.zeros((LANES,), jnp.float32)
        ixs, ixd, ixc = (ixs0, ixs1), (ixd0, ixd1), (ixc0, ixc1)
        zsb, zdb, stb = (zs0, zs1), (zd0, zd1), (st0, st1)
        gsem, ssem, isem = (gsem0, gsem1), (ssem0, ssem1), (isem0, isem1)

        def issue_idx(cid, b):
            pltpu.async_copy(src_hbm.at[wid].at[cid], ixs[b], isem[b])
            pltpu.async_copy(dst_hbm.at[wid].at[cid], ixd[b], isem[b])

        def wait_idx(cid, b):
            pltpu.make_async_copy(
                src_hbm.at[wid].at[cid], ixs[b], isem[b]).wait()
            pltpu.make_async_copy(
                dst_hbm.at[wid].at[cid], ixd[b], isem[b]).wait()

        def issue_gather(cid, b):
            pltpu.async_copy(z_hbm.at[ixs[b]], zsb[b], gsem[b])
            pltpu.async_copy(z_hbm.at[ixd[b]], zdb[b], gsem[b])

        def wait_gather(cid, b):
            pltpu.make_async_copy(z_hbm.at[ixs[b]], zsb[b], gsem[b]).wait()
            pltpu.make_async_copy(z_hbm.at[ixd[b]], zdb[b], gsem[b]).wait()

        def issue_scatter(cid, b):
            pltpu.async_copy(stb[b], acc.at[ixc[b]], ssem[b], add=True)

        def wait_scatter(cid, b):
            pltpu.make_async_copy(stb[b], acc.at[ixc[b]], ssem[b]).wait()

        issue_idx(0, 0)
        issue_idx(1, 1)

        @pl.loop(0, dr)
        def _(r):
            @pl.loop(0, 128, step=LANES)
            def _(j):
                den[r, pl.ds(j, LANES)] = zeros

        @pl.loop(0, dr, step=LANES)
        def _(r):
            idr[pl.ds(r, LANES)] = iota + r


        @pl.loop(0, ZROWS)
        def _(r):
            @pl.loop(0, 128, step=LANES)
            def _(j):
                st0[r, pl.ds(j, LANES)] = zeros

        @pl.loop(0, rows_per_tile, step=ZROWS)
        def _(r):
            pltpu.async_copy(st0.at[pl.ds(0, ZROWS)],
                             acc.at[pl.ds(s * rows_per_tile + r, ZROWS)],
                             zsem)

        @pl.when(s * 8 < dr)
        def _():
            pltpu.sync_copy(st0.at[pl.ds(0, 8)], accd.at[pl.ds(s * 8, 8)])

        @pl.loop(0, rows_per_tile, step=ZROWS)
        def _(r):
            pltpu.make_async_copy(
                st0.at[pl.ds(0, ZROWS)],
                acc.at[pl.ds(s * rows_per_tile + r, ZROWS)], zsem).wait()

        plsc.subcore_barrier()

        wait_idx(0, 0)
        issue_gather(0, 0)

        def compute(cid, b):
            zs, zd, stage, ixcb = zsb[b], zdb[b], stb[b], ixc[b]

            @pl.loop(0, CHUNK, step=LANES)
            def _(g):
                idxv = ixcb[pl.ds(g, LANES)]

                def tree(vals):
                    while len(vals) > 1:
                        vals = [vals[k] + vals[k + 1]
                                for k in range(0, len(vals) - 1, 2)] + (
                            [vals[-1]] if len(vals) % 2 else [])
                    return vals[0]

                NI = 4
                for l in range(0, LANES, NI):
                    ii = [g + l + t for t in range(NI)]
                    zsv, tot = [], []
                    for t in range(NI):
                        zrow = [zs[ii[t], pl.ds(16 * j, 16)]
                                for j in range(a // 16)]
                        drow = [zd[ii[t], pl.ds(16 * j, 16)]
                                for j in range(a // 16)]
                        zsv.append(zrow)
                        tot.append(tree([zrow[j] * drow[j]
                                         for j in range(a // 16)]))
                    tot = [(plsc.cumsum(v)
                            + jnp.flip(plsc.cumsum(jnp.flip(v)))) - v
                           for v in tot]
                    tot = [jnp.where(v > 0.0, v, 0.2 * v) for v in tot]
                    exv = [jnp.exp(jnp.minimum(v, 80.0)) for v in tot]
                    for t in range(NI):
                        for j in range(a // 16):
                            stage[ii[t], pl.ds(16 * j, 16)] = (
                                exv[t] * zsv[t][j])
                    for t in range(NI):
                        di = idxv[l + t]
                        r0 = lax.shift_right_logical(di, 7)
                        c0 = jnp.bitwise_and(di, 0x70)
                        lane = jnp.bitwise_and(di, 0xF)
                        upd = jnp.where(iota == lane, exv[t], 0.0)
                        den[r0, pl.ds(c0, LANES)] = (
                            den[r0, pl.ds(c0, LANES)] + upd)

        @pl.loop(0, ((nch + 1) // 2) * 2, step=2)
        def _(base_cid):
            for b in range(2):
                cid = base_cid + b

                def chunk_body():
                    wait_gather(cid, b)

                    @pl.when(cid >= 2)
                    def _():
                        wait_scatter(cid - 2, b)

                    @pl.loop(0, CHUNK, step=LANES)
                    def _(g):
                        ixc[b][pl.ds(g, LANES)] = ixd[b][pl.ds(g, LANES)]

                    @pl.when(cid + 2 < nch)
                    def _():
                        issue_idx(cid + 2, b)

                    @pl.when(cid + 1 < nch)
                    def _():
                        wait_idx(cid + 1, 1 - b)
                        issue_gather(cid + 1, 1 - b)

                    compute(cid, b)
                    issue_scatter(cid, b)

                if b == 0:
                    chunk_body()
                else:
                    pl.when(cid < nch)(chunk_body)

        wait_scatter(nch - 1, (nch - 1) % 2)
        wait_scatter(nch - 2, (nch - 2) % 2)

        pltpu.sync_copy(den, accd.at[idr], add=True)
        plsc.subcore_barrier()

        r0 = s * rows_per_tile
        pltpu.sync_copy(acc.at[pl.ds(r0, rows_per_tile)],
                        outh_hbm.at[c].at[pl.ds(r0, rows_per_tile)])

        @pl.when(s * 8 < dr)
        def _():
            pltpu.sync_copy(accd.at[pl.ds(s * 8, 8)],
                            outd_hbm.at[c].at[pl.ds(s * 8, 8)])

    return edge_kernel(src3d, dst3d, z)


def _finish_body(hs_ref, den_ref, out_ref):
    h = hs_ref[0] + hs_ref[1]
    den = den_ref[0] + den_ref[1]
    v = h / (den + 1e-16)
    out_ref[...] = jnp.where(v > 0, v, jnp.exp(jnp.minimum(v, 0.0)) - 1.0)


def _finish(hs, den, npad, a):
    blk = 2048
    return pl.pallas_call(
        _finish_body,
        grid=(npad // blk,),
        in_specs=[
            pl.BlockSpec((NC, blk, a), lambda i: (0, i, 0)),
            pl.BlockSpec((NC, blk, 1), lambda i: (0, i, 0)),
        ],
        out_specs=pl.BlockSpec((blk, a), lambda i: (i, 0)),
        out_shape=jax.ShapeDtypeStruct((npad, a), jnp.float32),
    )(hs, den)


def kernel(x, Wm, Wl, edge_index, node_type):
    n, d = x.shape
    a = Wm.shape[0]
    e = edge_index.shape[1]
    ntiles = NC * NS
    nch = -(-e // (ntiles * CHUNK))
    epad = ntiles * nch * CHUNK - e
    src = edge_index[0]
    dst = edge_index[1]
    if epad:
        fill = jnp.arange(epad, dtype=jnp.int32)
        src = jnp.concatenate([src, fill % n])
        dst = jnp.concatenate([dst, n + (fill % 128)])
    src3d = src.reshape(ntiles, nch, CHUNK)
    dst3d = dst.reshape(ntiles, nch, CHUNK)
    z = _compute_z(x, Wm, Wl, node_type.reshape(n, 1))
    hs, dend = _edge_pass(src3d, dst3d, z)
    npad = hs.shape[1]
    den = dend.reshape(NC, npad, 1)
    out = _finish(hs, den, npad, a)
    return out[:n]
